# baseline (speedup 1.0000x reference)
import functools

import jax
import jax.numpy as jnp
from jax import lax
from jax.experimental import pallas as pl
from jax.experimental.pallas import tpu as pltpu
from jax.experimental.pallas import tpu_sc as plsc

L = 16
ROW = 128
NC = 2
NS = 16


@functools.lru_cache(maxsize=None)
def _make_emb_kernel(V, D, NB, SEQ):
    NW = NC * NS
    iw = NB // NW
    K = iw // ROW
    group = K * ROW
    n_group = SEQ
    assert NB % (NW * ROW) == 0 and D == 2 * L and n_group % 2 == 0

    mesh = plsc.VectorSubcoreMesh(core_axis_name="c", subcore_axis_name="s")

    @functools.partial(
        pl.kernel,
        out_type=jax.ShapeDtypeStruct((SEQ, D, NB), jnp.float32),
        mesh=mesh,
        compiler_params=pltpu.CompilerParams(
            use_tc_tiling_on_sc=False, needs_layout_passes=False
        ),
        scratch_types=[
            pltpu.VMEM((SEQ * K, ROW), jnp.int32),
            pltpu.VMEM((group, D), jnp.float32),
            pltpu.VMEM((group, D), jnp.float32),
            pltpu.VMEM((D, group + 1), jnp.float32),
            pltpu.VMEM((D, group + 1), jnp.float32),
            pltpu.VMEM((L,), jnp.float32),
            pltpu.SemaphoreType.DMA,
            pltpu.SemaphoreType.DMA,
            pltpu.SemaphoreType.DMA,
            pltpu.SemaphoreType.DMA,
        ],
    )
    def emb(table_hbm, idx_hbm, s_hbm, out_hbm,
            idx_v, r0, r1, t0, t1, sv, g0, g1, o0, o1):
        wid = lax.axis_index("s") * NC + lax.axis_index("c")
        ibase = wid * iw
        pltpu.sync_copy(idx_hbm.at[wid], idx_v)
        pltpu.sync_copy(s_hbm, sv)
        s = jnp.exp(sv[...])
        riota = lax.iota(jnp.int32, L)

        rows = (r0, r1)
        tbuf = (t0, t1)
        gsem = (g0, g1)
        osem = (o0, o1)

        def fire(g, b):
            for k in range(K):
                pltpu.make_async_copy(
                    table_hbm.at[idx_v.at[g * K + k]],
                    rows[b].at[pl.ds(k * ROW, ROW)],
                    gsem[b],
                ).start()

        def drain(b):
            pltpu.make_async_copy(
                table_hbm.at[pl.ds(0, group)], rows[b], gsem[b]
            ).wait()

        def transpose_scale(b):
            r, t = rows[b], tbuf[b]
            hi = riota + L

            def body_i(i, _):
                col = jnp.full((L,), i, dtype=jnp.int32)
                v0 = r[i, pl.ds(0, L)]
                v1 = r[i, pl.ds(L, L)]
                plsc.store_scatter(t, [riota, col], v0 * s)
                plsc.store_scatter(t, [hi, col], v1 * s)
                return 0

            lax.fori_loop(0, group, body_i, 0, unroll=4)

        def flush(g, b):
            pltpu.make_async_copy(
                tbuf[b].at[:, pl.ds(0, group)],
                out_hbm.at[g, :, pl.ds(ibase, group)],
                osem[b],
            ).start()

        def flush_wait(b):
            pltpu.make_async_copy(
                tbuf[b].at[:, pl.ds(0, group)],
                out_hbm.at[0, :, pl.ds(ibase, group)],
                osem[b],
            ).wait()

        def step(g, b, *, prefetch=True, wait_out=True):
            if prefetch:
                fire(g + 1, 1 - b)
            drain(b)
            if wait_out:
                flush_wait(b)
            transpose_scale(b)
            flush(g, b)

        fire(0, 0)
        step(0, 0, wait_out=False)
        step(1, 1, wait_out=False)

        def outer(t, _):
            go = t * 2 + 2
            step(go, 0)
            step(go + 1, 1)
            return 0

        lax.fori_loop(0, (n_group - 4) // 2, outer, 0)
        step(n_group - 2, 0)
        step(n_group - 1, 1, prefetch=False)
        flush_wait(0)
        flush_wait(1)

    return emb, NW, K


@functools.lru_cache(maxsize=None)
def _make_table_transpose(V, D):
    NW = NC * NS
    n_full = V // ROW
    out_rows = V * D // ROW
    tail = V - n_full * ROW
    tail_rows = tail * D // ROW
    count = (n_full + NW - 1) // NW
    assert V % 8 == 0 and D == 2 * L and count % 2 == 1

    mesh = plsc.VectorSubcoreMesh(core_axis_name="c", subcore_axis_name="s")

    @functools.partial(
        pl.kernel,
        out_type=jax.ShapeDtypeStruct((out_rows, ROW), jnp.float32),
        mesh=mesh,
        compiler_params=pltpu.CompilerParams(
            use_tc_tiling_on_sc=True, needs_layout_passes=False
        ),
        scratch_types=[
            pltpu.VMEM((D, ROW + 1), jnp.float32),
            pltpu.VMEM((D, ROW + 1), jnp.float32),
            pltpu.VMEM((D, ROW), jnp.float32),
            pltpu.VMEM((tail_rows, ROW), jnp.float32),
            pltpu.SemaphoreType.DMA,
            pltpu.SemaphoreType.DMA,
        ],
    )
    def tkern(wt_hbm, tail_hbm, out_hbm, i0, i1, tout, tailbuf, s0, s1):
        wid = lax.axis_index("s") * NC + lax.axis_index("c")
        tin = (i0, i1)
        sem = (s0, s1)
        riota = lax.iota(jnp.int32, L)

        def col_of(k):
            c = wid + NW * k
            return jnp.where(c < n_full, c, 0)

        def fire(k, b):
            c = col_of(k)
            sc = pl.multiple_of(c * ROW, ROW)
            pltpu.make_async_copy(
                wt_hbm.at[:, pl.ds(sc, ROW)],
                tin[b].at[:, pl.ds(0, ROW)],
                sem[b],
            ).start()

        def drain(b):
            pltpu.make_async_copy(
                wt_hbm.at[:, pl.ds(0, ROW)],
                tin[b].at[:, pl.ds(0, ROW)],
                sem[b],
            ).wait()

        def transpose(b):
            t = tin[b]

            def body_a(a, _):
                for n in range(8):
                    rows_v = riota + (n % 2) * L
                    col = jnp.full((L,), a * 4 + n // 2, dtype=jnp.int32)
                    tout[a, pl.ds(n * L, L)] = plsc.load_gather(
                        t, [rows_v, col]
                    )
                return 0

            lax.fori_loop(0, D, body_a, 0)

        def flush(k):
            orow = pl.multiple_of(col_of(k) * D, D)
            pltpu.sync_copy(tout, out_hbm.at[pl.ds(orow, D)])

        def step(k, b, *, prefetch=True):
            if prefetch:
                fire(k + 1, 1 - b)
            drain(b)
            transpose(b)
            flush(k)

        pltpu.sync_copy(tail_hbm, tailbuf)
        pltpu.sync_copy(tailbuf, out_hbm.at[pl.ds(out_rows - tail_rows, tail_rows)])

        fire(0, 0)
        step(0, 0)

        def outer(t, _):
            ko = t * 2 + 1
            step(ko, 1)
            step(ko + 1, 0)
            return 0

        lax.fori_loop(0, (count - 1) // 2 - 1, outer, 0)
        step(count - 2, 1)
        step(count - 1, 0, prefetch=False)

    return tkern, n_full, tail


def kernel(input, weight, scale):
    V, D = weight.shape
    NB, SEQ = input.shape
    emb, NW, K = _make_emb_kernel(V, D, NB, SEQ)
    tkern, n_full, tail = _make_table_transpose(V, D)
    wtail = weight[n_full * ROW:, :].reshape(tail * D // ROW, ROW)
    wlin = tkern(weight.T, wtail).reshape(V, D)
    idxT = input.T.astype(jnp.int32)
    idx = (idxT.reshape(SEQ, NW, K, ROW)
           .transpose(1, 0, 2, 3)
           .reshape(NW, SEQ * K, ROW))
    svec = jnp.full((L,), scale, dtype=jnp.float32)
    outT = emb(wlin, idx, svec)
    return jnp.transpose(outT, (2, 0, 1))


# speedup vs baseline: 1.0459x; 1.0459x over previous
import functools

import jax
import jax.numpy as jnp
from jax import lax
from jax.experimental import pallas as pl
from jax.experimental.pallas import tpu as pltpu
from jax.experimental.pallas import tpu_sc as plsc

L = 16
ROW = 128
NC = 2
NS = 16


@functools.lru_cache(maxsize=None)
def _make_emb_kernel(V, D, NB, SEQ):
    NW = NC * NS
    iw = NB // NW
    K = iw // ROW
    group = K * ROW
    n_group = SEQ
    assert NB % (NW * ROW) == 0 and D == 2 * L and n_group % 2 == 0

    mesh = plsc.VectorSubcoreMesh(core_axis_name="c", subcore_axis_name="s")

    @functools.partial(
        pl.kernel,
        out_type=jax.ShapeDtypeStruct((SEQ, D, NB), jnp.float32),
        mesh=mesh,
        compiler_params=pltpu.CompilerParams(
            use_tc_tiling_on_sc=False, needs_layout_passes=False
        ),
        scratch_types=[
            pltpu.VMEM((SEQ * K, ROW), jnp.int32),
            pltpu.VMEM((group, D), jnp.float32),
            pltpu.VMEM((group, D), jnp.float32),
            pltpu.VMEM((D, group + 1), jnp.float32),
            pltpu.VMEM((D, group + 1), jnp.float32),
            pltpu.VMEM((L,), jnp.float32),
            pltpu.SemaphoreType.DMA,
            pltpu.SemaphoreType.DMA,
            pltpu.SemaphoreType.DMA,
            pltpu.SemaphoreType.DMA,
        ],
    )
    def emb(table_hbm, idx_hbm, s_hbm, out_hbm,
            idx_v, r0, r1, t0, t1, sv, g0, g1, o0, o1):
        wid = lax.axis_index("s") * NC + lax.axis_index("c")
        ibase = wid * iw
        pltpu.sync_copy(idx_hbm.at[wid], idx_v)
        pltpu.sync_copy(s_hbm, sv)
        s = jnp.exp(sv[...])
        riota = lax.iota(jnp.int32, L)

        rows = (r0, r1)
        tbuf = (t0, t1)
        gsem = (g0, g1)
        osem = (o0, o1)

        def fire(g, b):
            for k in range(K):
                pltpu.make_async_copy(
                    table_hbm.at[idx_v.at[g * K + k]],
                    rows[b].at[pl.ds(k * ROW, ROW)],
                    gsem[b],
                ).start()

        def drain(b):
            pltpu.make_async_copy(
                table_hbm.at[pl.ds(0, group)], rows[b], gsem[b]
            ).wait()

        def transpose_scale(b):
            r, t = rows[b], tbuf[b]
            hi = riota + L

            def body_i(i, _):
                col = jnp.full((L,), i, dtype=jnp.int32)
                v0 = r[i, pl.ds(0, L)]
                v1 = r[i, pl.ds(L, L)]
                plsc.store_scatter(t, [riota, col], v0 * s)
                plsc.store_scatter(t, [hi, col], v1 * s)
                return 0

            lax.fori_loop(0, group, body_i, 0, unroll=4)

        def flush(g, b):
            pltpu.make_async_copy(
                tbuf[b].at[:, pl.ds(0, group)],
                out_hbm.at[g, :, pl.ds(ibase, group)],
                osem[b],
            ).start()

        def flush_wait(b):
            pltpu.make_async_copy(
                tbuf[b].at[:, pl.ds(0, group)],
                out_hbm.at[0, :, pl.ds(ibase, group)],
                osem[b],
            ).wait()

        def step(g, b, *, prefetch=True, wait_out=True):
            if prefetch:
                fire(g + 1, 1 - b)
            drain(b)
            if wait_out:
                flush_wait(b)
            transpose_scale(b)
            flush(g, b)

        fire(0, 0)
        step(0, 0, wait_out=False)
        step(1, 1, wait_out=False)

        def outer(t, _):
            go = t * 2 + 2
            step(go, 0)
            step(go + 1, 1)
            return 0

        lax.fori_loop(0, (n_group - 4) // 2, outer, 0)
        step(n_group - 2, 0)
        step(n_group - 1, 1, prefetch=False)
        flush_wait(0)
        flush_wait(1)

    return emb, NW, K


@functools.lru_cache(maxsize=None)
def _make_table_transpose(V, D):
    NW = NC * NS
    CC = 2
    CW = CC * ROW
    n_full = V // ROW
    n_chunk = n_full // CC
    out_rows = V * D // ROW
    tail = V - n_full * ROW
    tail_rows = tail * D // ROW
    count = (n_chunk + NW - 1) // NW
    assert V % 8 == 0 and D == 2 * L and n_full % CC == 0
    assert count % 2 == 1 and count >= 5

    mesh = plsc.VectorSubcoreMesh(core_axis_name="c", subcore_axis_name="s")

    @functools.partial(
        pl.kernel,
        out_type=jax.ShapeDtypeStruct((out_rows, ROW), jnp.float32),
        mesh=mesh,
        compiler_params=pltpu.CompilerParams(
            use_tc_tiling_on_sc=True, needs_layout_passes=False
        ),
        scratch_types=[
            pltpu.VMEM((D, CW + 1), jnp.float32),
            pltpu.VMEM((D, CW + 1), jnp.float32),
            pltpu.VMEM((CC * D, ROW), jnp.float32),
            pltpu.VMEM((CC * D, ROW), jnp.float32),
            pltpu.VMEM((tail_rows, ROW), jnp.float32),
            pltpu.SemaphoreType.DMA,
            pltpu.SemaphoreType.DMA,
            pltpu.SemaphoreType.DMA,
            pltpu.SemaphoreType.DMA,
        ],
    )
    def tkern(wt_hbm, tail_hbm, out_hbm,
              i0, i1, t0, t1, tailbuf, s0, s1, o0, o1):
        wid = lax.axis_index("s") * NC + lax.axis_index("c")
        tin = (i0, i1)
        tout = (t0, t1)
        sem = (s0, s1)
        osem = (o0, o1)
        riota = lax.iota(jnp.int32, L)

        def chunk_of(k):
            q = wid + NW * k
            return jnp.where(q < n_chunk, q, 0)

        def fire(k, b):
            sc = pl.multiple_of(chunk_of(k) * CW, CW)
            pltpu.make_async_copy(
                wt_hbm.at[:, pl.ds(sc, CW)],
                tin[b].at[:, pl.ds(0, CW)],
                sem[b],
            ).start()

        def drain(b):
            pltpu.make_async_copy(
                wt_hbm.at[:, pl.ds(0, CW)],
                tin[b].at[:, pl.ds(0, CW)],
                sem[b],
            ).wait()

        def transpose(b):
            t = tin[b], tout[b]

            def body_a(a, _):
                for n in range(8):
                    rows_v = riota + (n % 2) * L
                    col = jnp.full((L,), a * 4 + n // 2, dtype=jnp.int32)
                    t[1][a, pl.ds(n * L, L)] = plsc.load_gather(
                        t[0], [rows_v, col]
                    )
                return 0

            lax.fori_loop(0, CC * D, body_a, 0)

        def flush(k, b):
            orow = pl.multiple_of(chunk_of(k) * CC * D, CC * D)
            pltpu.make_async_copy(
                tout[b], out_hbm.at[pl.ds(orow, CC * D)], osem[b]
            ).start()

        def flush_wait(b):
            pltpu.make_async_copy(
                tout[b], out_hbm.at[pl.ds(0, CC * D)], osem[b]
            ).wait()

        def step(k, b, *, prefetch=True, wait_out=True):
            if prefetch:
                fire(k + 1, 1 - b)
            drain(b)
            if wait_out:
                flush_wait(b)
            transpose(b)
            flush(k, b)

        pltpu.sync_copy(tail_hbm, tailbuf)
        pltpu.sync_copy(
            tailbuf, out_hbm.at[pl.ds(out_rows - tail_rows, tail_rows)]
        )

        fire(0, 0)
        step(0, 0, wait_out=False)
        step(1, 1, wait_out=False)
        step(2, 0)

        def outer(t, _):
            ko = t * 2 + 3
            step(ko, 1)
            step(ko + 1, 0)
            return 0

        lax.fori_loop(0, (count - 5) // 2, outer, 0)
        step(count - 2, 1)
        step(count - 1, 0, prefetch=False)
        flush_wait(1)
        flush_wait(0)

    return tkern, n_full, tail


def kernel(input, weight, scale):
    V, D = weight.shape
    NB, SEQ = input.shape
    emb, NW, K = _make_emb_kernel(V, D, NB, SEQ)
    tkern, n_full, tail = _make_table_transpose(V, D)
    wtail = weight[n_full * ROW:, :].reshape(tail * D // ROW, ROW)
    wlin = tkern(weight.T, wtail).reshape(V, D)
    idxT = input.T.astype(jnp.int32)
    idx = (idxT.reshape(SEQ, NW, K, ROW)
           .transpose(1, 0, 2, 3)
           .reshape(NW, SEQ * K, ROW))
    svec = jnp.full((L,), scale, dtype=jnp.float32)
    outT = emb(wlin, idx, svec)
    return jnp.transpose(outT, (2, 0, 1))


# speedup vs baseline: 1.3707x; 1.3105x over previous
import functools

import jax
import jax.numpy as jnp
from jax import lax
from jax.experimental import pallas as pl
from jax.experimental.pallas import tpu as pltpu
from jax.experimental.pallas import tpu_sc as plsc

L = 16
ROW = 128
NC = 2
NS = 16


@functools.lru_cache(maxsize=None)
def _make_emb_kernel(V, D, NB, SEQ):
    NW = NC * NS
    iw = NB // NW
    K = iw // ROW
    group = K * ROW
    n_group = SEQ
    assert NB % (NW * ROW) == 0 and D == 2 * L and n_group % 2 == 0

    mesh = plsc.VectorSubcoreMesh(core_axis_name="c", subcore_axis_name="s")

    @functools.partial(
        pl.kernel,
        out_type=jax.ShapeDtypeStruct((SEQ, D, NB), jnp.float32),
        mesh=mesh,
        compiler_params=pltpu.CompilerParams(
            use_tc_tiling_on_sc=False, needs_layout_passes=False
        ),
        scratch_types=[
            pltpu.VMEM((SEQ * K, ROW), jnp.int32),
            pltpu.VMEM((group, D), jnp.float32),
            pltpu.VMEM((group, D), jnp.float32),
            pltpu.VMEM((D, group + 1), jnp.float32),
            pltpu.VMEM((D, group + 1), jnp.float32),
            pltpu.VMEM((L,), jnp.float32),
            pltpu.SemaphoreType.DMA,
            pltpu.SemaphoreType.DMA,
            pltpu.SemaphoreType.DMA,
            pltpu.SemaphoreType.DMA,
        ],
    )
    def emb(table_hbm, idx_hbm, s_hbm, out_hbm,
            idx_v, r0, r1, t0, t1, sv, g0, g1, o0, o1):
        wid = lax.axis_index("s") * NC + lax.axis_index("c")
        ibase = wid * iw
        pltpu.sync_copy(idx_hbm.at[wid], idx_v)
        pltpu.sync_copy(s_hbm, sv)
        s = jnp.exp(sv[...])
        riota = lax.iota(jnp.int32, L)

        rows = (r0, r1)
        tbuf = (t0, t1)
        gsem = (g0, g1)
        osem = (o0, o1)

        def fire(g, b):
            for k in range(K):
                pltpu.make_async_copy(
                    table_hbm.at[idx_v.at[g * K + k]],
                    rows[b].at[pl.ds(k * ROW, ROW)],
                    gsem[b],
                ).start()

        def drain(b):
            pltpu.make_async_copy(
                table_hbm.at[pl.ds(0, group)], rows[b], gsem[b]
            ).wait()

        def transpose_scale(b):
            r, t = rows[b], tbuf[b]
            hi = riota + L

            def body_i(i, _):
                col = jnp.full((L,), i, dtype=jnp.int32)
                v0 = r[i, pl.ds(0, L)]
                v1 = r[i, pl.ds(L, L)]
                plsc.store_scatter(t, [riota, col], v0 * s)
                plsc.store_scatter(t, [hi, col], v1 * s)
                return 0

            lax.fori_loop(0, group, body_i, 0, unroll=4)

        def flush(g, b):
            pltpu.make_async_copy(
                tbuf[b].at[:, pl.ds(0, group)],
                out_hbm.at[g, :, pl.ds(ibase, group)],
                osem[b],
            ).start()

        def flush_wait(b):
            pltpu.make_async_copy(
                tbuf[b].at[:, pl.ds(0, group)],
                out_hbm.at[0, :, pl.ds(ibase, group)],
                osem[b],
            ).wait()

        def step(g, b, *, prefetch=True, wait_out=True):
            if prefetch:
                fire(g + 1, 1 - b)
            drain(b)
            if wait_out:
                flush_wait(b)
            transpose_scale(b)
            flush(g, b)

        fire(0, 0)
        step(0, 0, wait_out=False)
        step(1, 1, wait_out=False)

        def outer(t, _):
            go = t * 2 + 2
            step(go, 0)
            step(go + 1, 1)
            return 0

        lax.fori_loop(0, (n_group - 4) // 2, outer, 0)
        step(n_group - 2, 0)
        step(n_group - 1, 1, prefetch=False)
        flush_wait(0)
        flush_wait(1)

    return emb, NW, K


@functools.lru_cache(maxsize=None)
def _make_table_transpose(V, D):
    NW = NC * NS
    CC = 2
    CW = CC * ROW
    n_full = V // ROW
    n_chunk = n_full // CC
    out_rows = V * D // ROW
    tail = V - n_full * ROW
    tail_rows = tail * D // ROW
    count = (n_chunk + NW - 1) // NW
    assert V % 8 == 0 and D == 2 * L and n_full % CC == 0
    assert count % 2 == 1 and count >= 5

    mesh = plsc.VectorSubcoreMesh(core_axis_name="c", subcore_axis_name="s")

    @functools.partial(
        pl.kernel,
        out_type=jax.ShapeDtypeStruct((out_rows, ROW), jnp.float32),
        mesh=mesh,
        compiler_params=pltpu.CompilerParams(
            use_tc_tiling_on_sc=True, needs_layout_passes=False
        ),
        scratch_types=[
            pltpu.VMEM((D, CW + 1), jnp.float32),
            pltpu.VMEM((D, CW + 1), jnp.float32),
            pltpu.VMEM((CC * D, ROW), jnp.float32),
            pltpu.VMEM((CC * D, ROW), jnp.float32),
            pltpu.VMEM((tail_rows, ROW), jnp.float32),
            pltpu.SemaphoreType.DMA,
            pltpu.SemaphoreType.DMA,
            pltpu.SemaphoreType.DMA,
            pltpu.SemaphoreType.DMA,
        ],
    )
    def tkern(wt_hbm, tail_hbm, out_hbm,
              i0, i1, t0, t1, tailbuf, s0, s1, o0, o1):
        wid = lax.axis_index("s") * NC + lax.axis_index("c")
        tin = (i0, i1)
        tout = (t0, t1)
        sem = (s0, s1)
        osem = (o0, o1)
        riota = lax.iota(jnp.int32, L)

        def chunk_of(k):
            q = wid + NW * k
            return jnp.where(q < n_chunk, q, 0)

        def fire(k, b):
            sc = pl.multiple_of(chunk_of(k) * CW, CW)
            pltpu.make_async_copy(
                wt_hbm.at[:, pl.ds(sc, CW)],
                tin[b].at[:, pl.ds(0, CW)],
                sem[b],
            ).start()

        def drain(b):
            pltpu.make_async_copy(
                wt_hbm.at[:, pl.ds(0, CW)],
                tin[b].at[:, pl.ds(0, CW)],
                sem[b],
            ).wait()

        def transpose(b):
            t = tin[b], tout[b]

            def body_a(a, _):
                vs = []
                for n in range(8):
                    rows_v = riota + (n % 2) * L
                    col = jnp.full((L,), a * 4 + n // 2, dtype=jnp.int32)
                    vs.append(plsc.load_gather(t[0], [rows_v, col]))
                for n in range(8):
                    t[1][a, pl.ds(n * L, L)] = vs[n]
                return 0

            lax.fori_loop(0, CC * D, body_a, 0, unroll=2)

        def flush(k, b):
            orow = pl.multiple_of(chunk_of(k) * CC * D, CC * D)
            pltpu.make_async_copy(
                tout[b], out_hbm.at[pl.ds(orow, CC * D)], osem[b]
            ).start()

        def flush_wait(b):
            pltpu.make_async_copy(
                tout[b], out_hbm.at[pl.ds(0, CC * D)], osem[b]
            ).wait()

        def step(k, b, *, prefetch=True, wait_out=True):
            if prefetch:
                fire(k + 1, 1 - b)
            drain(b)
            if wait_out:
                flush_wait(b)
            transpose(b)
            flush(k, b)

        pltpu.sync_copy(tail_hbm, tailbuf)
        pltpu.sync_copy(
            tailbuf, out_hbm.at[pl.ds(out_rows - tail_rows, tail_rows)]
        )

        fire(0, 0)
        step(0, 0, wait_out=False)
        step(1, 1, wait_out=False)
        step(2, 0)

        def outer(t, _):
            ko = t * 2 + 3
            step(ko, 1)
            step(ko + 1, 0)
            return 0

        lax.fori_loop(0, (count - 5) // 2, outer, 0)
        step(count - 2, 1)
        step(count - 1, 0, prefetch=False)
        flush_wait(1)
        flush_wait(0)

    return tkern, n_full, tail


def kernel(input, weight, scale):
    V, D = weight.shape
    NB, SEQ = input.shape
    emb, NW, K = _make_emb_kernel(V, D, NB, SEQ)
    tkern, n_full, tail = _make_table_transpose(V, D)
    wtail = weight[n_full * ROW:, :].reshape(tail * D // ROW, ROW)
    wlin = tkern(weight.T, wtail).reshape(V, D)
    idxT = input.T.astype(jnp.int32)
    idx = (idxT.reshape(SEQ, NW, K, ROW)
           .transpose(1, 0, 2, 3)
           .reshape(NW, SEQ * K, ROW))
    svec = jnp.full((L,), scale, dtype=jnp.float32)
    outT = emb(wlin, idx, svec)
    return jnp.transpose(outT, (2, 0, 1))


# speedup vs baseline: 1.4882x; 1.0857x over previous
import functools

import jax
import jax.numpy as jnp
from jax import lax
from jax.experimental import pallas as pl
from jax.experimental.pallas import tpu as pltpu
from jax.experimental.pallas import tpu_sc as plsc

L = 16
ROW = 128
NC = 2
NS = 16


@functools.lru_cache(maxsize=None)
def _make_emb_kernel(V, D, NB, SEQ):
    NW = NC * NS
    iw = NB // NW
    K = iw // ROW
    group = K * ROW
    n_group = SEQ
    assert NB % (NW * ROW) == 0 and D == 2 * L and n_group % 2 == 0

    mesh = plsc.VectorSubcoreMesh(core_axis_name="c", subcore_axis_name="s")

    @functools.partial(
        pl.kernel,
        out_type=jax.ShapeDtypeStruct((SEQ, D, NB), jnp.float32),
        mesh=mesh,
        compiler_params=pltpu.CompilerParams(
            use_tc_tiling_on_sc=False, needs_layout_passes=False
        ),
        scratch_types=[
            pltpu.VMEM((SEQ * K, ROW), jnp.int32),
            pltpu.VMEM((group, D), jnp.float32),
            pltpu.VMEM((group, D), jnp.float32),
            pltpu.VMEM((D, group + 1), jnp.float32),
            pltpu.VMEM((D, group + 1), jnp.float32),
            pltpu.VMEM((L,), jnp.float32),
            pltpu.SemaphoreType.DMA,
            pltpu.SemaphoreType.DMA,
            pltpu.SemaphoreType.DMA,
            pltpu.SemaphoreType.DMA,
        ],
    )
    def emb(table_hbm, idx_hbm, s_hbm, out_hbm,
            idx_v, r0, r1, t0, t1, sv, g0, g1, o0, o1):
        wid = lax.axis_index("s") * NC + lax.axis_index("c")
        ibase = wid * iw
        pltpu.sync_copy(idx_hbm.at[wid], idx_v)
        pltpu.sync_copy(s_hbm, sv)
        s = jnp.exp(sv[...])
        riota = lax.iota(jnp.int32, L)

        rows = (r0, r1)
        tbuf = (t0, t1)
        gsem = (g0, g1)
        osem = (o0, o1)

        def fire(g, b):
            for k in range(K):
                pltpu.make_async_copy(
                    table_hbm.at[idx_v.at[g * K + k]],
                    rows[b].at[pl.ds(k * ROW, ROW)],
                    gsem[b],
                ).start()

        def drain(b):
            pltpu.make_async_copy(
                table_hbm.at[pl.ds(0, group)], rows[b], gsem[b]
            ).wait()

        def transpose_scale(b):
            r, t = rows[b], tbuf[b]
            hi = riota + L

            def body_i(i, _):
                col = jnp.full((L,), i, dtype=jnp.int32)
                v0 = r[i, pl.ds(0, L)]
                v1 = r[i, pl.ds(L, L)]
                plsc.store_scatter(t, [riota, col], v0 * s)
                plsc.store_scatter(t, [hi, col], v1 * s)
                return 0

            lax.fori_loop(0, group, body_i, 0, unroll=4)

        def flush(g, b):
            pltpu.make_async_copy(
                tbuf[b].at[:, pl.ds(0, group)],
                out_hbm.at[g, :, pl.ds(ibase, group)],
                osem[b],
            ).start()

        def flush_wait(b):
            pltpu.make_async_copy(
                tbuf[b].at[:, pl.ds(0, group)],
                out_hbm.at[0, :, pl.ds(ibase, group)],
                osem[b],
            ).wait()

        def step(g, b, *, prefetch=True, wait_out=True):
            if prefetch:
                fire(g + 1, 1 - b)
            drain(b)
            if wait_out:
                flush_wait(b)
            transpose_scale(b)
            flush(g, b)

        fire(0, 0)
        step(0, 0, wait_out=False)
        step(1, 1, wait_out=False)

        def outer(t, _):
            go = t * 2 + 2
            step(go, 0)
            step(go + 1, 1)
            return 0

        lax.fori_loop(0, (n_group - 4) // 2, outer, 0)
        step(n_group - 2, 0)
        step(n_group - 1, 1, prefetch=False)
        flush_wait(0)
        flush_wait(1)

    return emb, NW, K


@functools.lru_cache(maxsize=None)
def _make_table_transpose(V, D):
    NW = NC * NS
    CC = 2
    CW = CC * ROW
    n_full = V // ROW
    n_chunk = n_full // CC
    out_rows = V * D // ROW
    tail = V - n_full * ROW
    tail_rows = tail * D // ROW
    count = (n_chunk + NW - 1) // NW
    assert V % 8 == 0 and D == 2 * L and n_full % CC == 0
    assert count % 2 == 1 and count >= 5

    mesh = plsc.VectorSubcoreMesh(core_axis_name="c", subcore_axis_name="s")

    @functools.partial(
        pl.kernel,
        out_type=jax.ShapeDtypeStruct((out_rows, ROW), jnp.float32),
        mesh=mesh,
        compiler_params=pltpu.CompilerParams(
            use_tc_tiling_on_sc=True, needs_layout_passes=False
        ),
        scratch_types=[
            pltpu.VMEM((D, CW + 1), jnp.float32),
            pltpu.VMEM((D, CW + 1), jnp.float32),
            pltpu.VMEM((CC * D, ROW), jnp.float32),
            pltpu.VMEM((CC * D, ROW), jnp.float32),
            pltpu.VMEM((tail_rows, ROW), jnp.float32),
            pltpu.SemaphoreType.DMA,
            pltpu.SemaphoreType.DMA,
            pltpu.SemaphoreType.DMA,
            pltpu.SemaphoreType.DMA,
        ],
    )
    def tkern(wt_hbm, tail_hbm, out_hbm,
              i0, i1, t0, t1, tailbuf, s0, s1, o0, o1):
        wid = lax.axis_index("s") * NC + lax.axis_index("c")
        tin = (i0, i1)
        tout = (t0, t1)
        sem = (s0, s1)
        osem = (o0, o1)
        riota = lax.iota(jnp.int32, L)

        def chunk_of(k):
            q = wid + NW * k
            return jnp.where(q < n_chunk, q, 0)

        def fire(k, b):
            sc = pl.multiple_of(chunk_of(k) * CW, CW)
            pltpu.make_async_copy(
                wt_hbm.at[:, pl.ds(sc, CW)],
                tin[b].at[:, pl.ds(0, CW)],
                sem[b],
            ).start()

        def drain(b):
            pltpu.make_async_copy(
                wt_hbm.at[:, pl.ds(0, CW)],
                tin[b].at[:, pl.ds(0, CW)],
                sem[b],
            ).wait()

        rlo = riota
        rhi = riota + L

        def transpose(b):
            t = tin[b], tout[b]

            def body_a(a, _):
                c0 = jnp.full((L,), a * 4, dtype=jnp.int32)
                cols = [c0, c0 + 1, c0 + 2, c0 + 3]
                vs = []
                for n in range(8):
                    rows_v = rhi if n % 2 else rlo
                    vs.append(plsc.load_gather(t[0], [rows_v, cols[n // 2]]))
                for n in range(8):
                    t[1][a, pl.ds(n * L, L)] = vs[n]
                return 0

            lax.fori_loop(0, CC * D, body_a, 0, unroll=2)

        def flush(k, b):
            orow = pl.multiple_of(chunk_of(k) * CC * D, CC * D)
            pltpu.make_async_copy(
                tout[b], out_hbm.at[pl.ds(orow, CC * D)], osem[b]
            ).start()

        def flush_wait(b):
            pltpu.make_async_copy(
                tout[b], out_hbm.at[pl.ds(0, CC * D)], osem[b]
            ).wait()

        def step(k, b, *, prefetch=True, wait_out=True):
            if prefetch:
                fire(k + 1, 1 - b)
            drain(b)
            if wait_out:
                flush_wait(b)
            transpose(b)
            flush(k, b)

        pltpu.sync_copy(tail_hbm, tailbuf)
        pltpu.sync_copy(
            tailbuf, out_hbm.at[pl.ds(out_rows - tail_rows, tail_rows)]
        )

        fire(0, 0)
        step(0, 0, wait_out=False)
        step(1, 1, wait_out=False)
        step(2, 0)

        def outer(t, _):
            ko = t * 2 + 3
            step(ko, 1)
            step(ko + 1, 0)
            return 0

        lax.fori_loop(0, (count - 5) // 2, outer, 0)
        step(count - 2, 1)
        step(count - 1, 0, prefetch=False)
        flush_wait(1)
        flush_wait(0)

    return tkern, n_full, tail


def kernel(input, weight, scale):
    V, D = weight.shape
    NB, SEQ = input.shape
    emb, NW, K = _make_emb_kernel(V, D, NB, SEQ)
    tkern, n_full, tail = _make_table_transpose(V, D)
    wtail = weight[n_full * ROW:, :].reshape(tail * D // ROW, ROW)
    wlin = tkern(weight.T, wtail).reshape(V, D)
    idxT = input.T.astype(jnp.int32)
    idx = (idxT.reshape(SEQ, NW, K, ROW)
           .transpose(1, 0, 2, 3)
           .reshape(NW, SEQ * K, ROW))
    svec = jnp.full((L,), scale, dtype=jnp.float32)
    outT = emb(wlin, idx, svec)
    return jnp.transpose(outT, (2, 0, 1))
